# baseline (speedup 1.0000x reference)
import functools

import jax
import jax.numpy as jnp
from jax import lax
from jax.experimental import pallas as pl
from jax.experimental.pallas import tpu as pltpu
from jax.experimental.pallas import tpu_sc as plsc

N_ROWS = 1000000
D = 32
B = 16384

_info = plsc.get_sparse_core_info()
NC = _info.num_cores
NS = _info.num_subcores
NW = NC * NS
B_PER_W = B // NW
CHUNK = 128
N_CHUNKS = B_PER_W // CHUNK


def _gather_body(table_hbm, idx_hbm, out_hbm, idx_v, rows_v, sem):
    wid = lax.axis_index("s") * NC + lax.axis_index("c")
    base = wid * B_PER_W
    pltpu.sync_copy(idx_hbm.at[wid], idx_v)
    copies = [
        pltpu.async_copy(
            table_hbm.at[idx_v.at[j]],
            rows_v.at[pl.ds(j * CHUNK, CHUNK)],
            sem,
        )
        for j in range(N_CHUNKS)
    ]
    for c in copies:
        c.wait()
    pltpu.sync_copy(rows_v, out_hbm.at[pl.ds(base, B_PER_W)])


@functools.partial(jax.jit, static_argnames=())
def _run(idx3, table):
    mesh = plsc.VectorSubcoreMesh(core_axis_name="c", subcore_axis_name="s")
    return pl.kernel(
        _gather_body,
        out_type=jax.ShapeDtypeStruct((B, D), jnp.float32),
        mesh=mesh,
        scratch_types=[
            pltpu.VMEM((N_CHUNKS, CHUNK), jnp.int32),
            pltpu.VMEM((B_PER_W, D), jnp.float32),
            pltpu.SemaphoreType.DMA,
        ],
        compiler_params=pltpu.CompilerParams(use_tc_tiling_on_sc=False),
    )(table, idx3)


def kernel(x, table):
    idx3 = x.astype(jnp.int32).reshape(NW, N_CHUNKS, CHUNK)
    return _run(idx3, table)


# speedup vs baseline: 3.3870x; 3.3870x over previous
import jax
import jax.numpy as jnp
from jax import lax
from jax.experimental import pallas as pl
from jax.experimental.pallas import tpu as pltpu
from jax.experimental.pallas import tpu_sc as plsc

N_ROWS = 1000000
D = 32
B = 16384
BLK = 128

_info = plsc.get_sparse_core_info()
NC = _info.num_cores
NS = _info.num_subcores
NW = NC * NS
B_PER_W = B // NW
L = 16
WAVES = B_PER_W // L


def _gather_body(table_t, idx_hbm, out1d, xv, ring, out_stage, sem):
    wid = lax.axis_index("s") * NC + lax.axis_index("c")
    base = wid * B_PER_W
    pltpu.sync_copy(idx_hbm.at[pl.ds(base, B_PER_W)], xv)

    iota = lax.iota(jnp.int32, L)
    zeros = jnp.zeros((L,), jnp.int32)

    def wave(g, _):
        v = xv[pl.ds(g * L, L)]
        bs = []
        copies = []
        for l in range(L):
            b = jnp.max(jnp.where(iota == l, v, 0), axis=0)
            blk = lax.shift_left(lax.shift_right_logical(b, 7), 7)
            bs.append((b, blk))
            copies.append(pltpu.async_copy(
                table_t.at[:, pl.ds(pl.multiple_of(blk, BLK), BLK)],
                ring.at[l],
                sem,
            ))
        for cp in copies:
            cp.wait()
        for l in range(L):
            b, blk = bs[l]
            lane = zeros + (b - blk)
            slot = zeros + l
            lo = plsc.load_gather(ring, (slot, iota, lane))
            hi = plsc.load_gather(ring, (slot, iota + L, lane))
            i = g * L + l
            out_stage[pl.ds(i * D, L)] = lo
            out_stage[pl.ds(i * D + L, L)] = hi
        return 0

    lax.fori_loop(0, WAVES, wave, 0)
    pltpu.sync_copy(out_stage, out1d.at[pl.ds(base * D, B_PER_W * D)])


@jax.jit
def _run(x, table_t):
    mesh = plsc.VectorSubcoreMesh(core_axis_name="c", subcore_axis_name="s")
    return pl.kernel(
        _gather_body,
        out_type=jax.ShapeDtypeStruct((B * D,), jnp.float32),
        mesh=mesh,
        scratch_types=[
            pltpu.VMEM((B_PER_W,), jnp.int32),
            pltpu.VMEM((L, D, BLK), jnp.float32),
            pltpu.VMEM((B_PER_W * D,), jnp.float32),
            pltpu.SemaphoreType.DMA,
        ],
        compiler_params=pltpu.CompilerParams(
            use_tc_tiling_on_sc=True,
            needs_layout_passes=False,
            disable_bounds_checks=True,
        ),
    )(table_t, x)


def kernel(x, table):
    out1d = _run(x.astype(jnp.int32), table.T)
    return out1d.reshape(B, D)


# speedup vs baseline: 4.0723x; 1.2023x over previous
import jax
import jax.numpy as jnp
from jax import lax
from jax.experimental import pallas as pl
from jax.experimental.pallas import tpu as pltpu
from jax.experimental.pallas import tpu_sc as plsc

N_ROWS = 1000000
D = 32
B = 16384
BLK = 128

_info = plsc.get_sparse_core_info()
NC = _info.num_cores
NS = _info.num_subcores
NW = NC * NS
B_PER_W = B // NW
L = 16
WAVES = B_PER_W // L


def _gather_body(table_t, idx_hbm, out_t, xv, ring, stage, sems):
    wid = lax.axis_index("s") * NC + lax.axis_index("c")
    base = wid * B_PER_W
    pltpu.sync_copy(idx_hbm.at[pl.ds(base, B_PER_W)], xv)

    iota = lax.iota(jnp.int32, L)
    zeros = jnp.zeros((L,), jnp.int32)

    def extract(v, l):
        return jnp.max(jnp.where(iota == l, v, 0), axis=0)

    def fire(b, l):
        blk = lax.shift_left(lax.shift_right_logical(b, 7), 7)
        pltpu.async_copy(
            table_t.at[:, pl.ds(pl.multiple_of(blk, BLK), BLK)],
            ring.at[l],
            sems.at[l],
        )
        return blk

    v0 = xv[pl.ds(0, L)]
    blks0 = [fire(extract(v0, l), l) for l in range(L)]
    del blks0

    def step(g, v_prev):
        off = jnp.minimum(g + 1, WAVES - 1) * L
        v_next = xv[pl.ds(off, L)]
        for l in range(L):
            b_p = extract(v_prev, l)
            blk_p = lax.shift_left(lax.shift_right_logical(b_p, 7), 7)
            pltpu.make_async_copy(
                table_t.at[:, pl.ds(0, BLK)], ring.at[l], sems.at[l]
            ).wait()
            lane = zeros + (b_p - blk_p)
            slot = zeros + l
            lo = plsc.load_gather(ring, (slot, iota, lane))
            hi = plsc.load_gather(ring, (slot, iota + L, lane))
            i = g * L + l
            plsc.store_scatter(stage, (iota, zeros + i), lo)
            plsc.store_scatter(stage, (iota + L, zeros + i), hi)

            b_n = extract(v_next, l)

            @pl.when(g < WAVES - 1)
            def _():
                fire(b_n, l)

        return v_next

    lax.fori_loop(0, WAVES, step, v0)
    pltpu.sync_copy(stage, out_t.at[:, pl.ds(base, B_PER_W)])


@jax.jit
def _run(x, table_t):
    mesh = plsc.VectorSubcoreMesh(core_axis_name="c", subcore_axis_name="s")
    return pl.kernel(
        _gather_body,
        out_type=jax.ShapeDtypeStruct((D, B), jnp.float32),
        mesh=mesh,
        scratch_types=[
            pltpu.VMEM((B_PER_W,), jnp.int32),
            pltpu.VMEM((L, D, BLK), jnp.float32),
            pltpu.VMEM((D, B_PER_W), jnp.float32),
            pltpu.SemaphoreType.DMA((L,)),
        ],
        compiler_params=pltpu.CompilerParams(
            use_tc_tiling_on_sc=True,
            needs_layout_passes=False,
            disable_bounds_checks=True,
        ),
    )(table_t, x)


def kernel(x, table):
    out_t = _run(x.astype(jnp.int32), table.T)
    return out_t.T


# speedup vs baseline: 4.1189x; 1.0114x over previous
import jax
import jax.numpy as jnp
from jax import lax
from jax.experimental import pallas as pl
from jax.experimental.pallas import tpu as pltpu
from jax.experimental.pallas import tpu_sc as plsc

N_ROWS = 1000000
D = 32
B = 16384
BLK = 128

_info = plsc.get_sparse_core_info()
NC = _info.num_cores
NS = _info.num_subcores
NW = NC * NS
B_PER_W = B // NW
L = 16
WAVES = B_PER_W // L


def _gather_body(table_3, idx_hbm, out_t, xv, ring, stage, sems):
    wid = lax.axis_index("s") * NC + lax.axis_index("c")
    base = wid * B_PER_W
    pltpu.sync_copy(idx_hbm.at[pl.ds(base, B_PER_W)], xv)

    iota = lax.iota(jnp.int32, L)
    zeros = jnp.zeros((L,), jnp.int32)
    jhi = lax.shift_right_logical(iota, 3)
    jlo = lax.bitwise_and(iota, 7)

    def extract(v, l):
        return jnp.max(jnp.where(iota == l, v, 0), axis=0)

    def fire(b, l):
        blk = lax.shift_left(lax.shift_right_logical(b, 7), 7)
        for jh in range(4):
            pltpu.async_copy(
                table_3.at[jh, :, pl.ds(pl.multiple_of(blk, BLK), BLK)],
                ring.at[l, jh],
                sems.at[l],
            )
        return blk

    v0 = xv[pl.ds(0, L)]
    for l in range(L):
        fire(extract(v0, l), l)

    def step(g, v_prev):
        off = jnp.minimum(g + 1, WAVES - 1) * L
        v_next = xv[pl.ds(off, L)]
        for l in range(L):
            b_p = extract(v_prev, l)
            blk_p = lax.shift_left(lax.shift_right_logical(b_p, 7), 7)
            pltpu.make_async_copy(
                table_3.at[:, :, pl.ds(0, BLK)], ring.at[l], sems.at[l]
            ).wait()
            lane = zeros + (b_p - blk_p)
            slot = zeros + l
            lo = plsc.load_gather(ring, (slot, jhi, jlo, lane))
            hi = plsc.load_gather(ring, (slot, jhi + 2, jlo, lane))
            i = g * L + l
            plsc.store_scatter(stage, (iota, zeros + i), lo)
            plsc.store_scatter(stage, (iota + L, zeros + i), hi)

            b_n = extract(v_next, l)

            @pl.when(g < WAVES - 1)
            def _():
                fire(b_n, l)

        return v_next

    lax.fori_loop(0, WAVES, step, v0)
    pltpu.sync_copy(stage, out_t.at[:, pl.ds(base, B_PER_W)])


@jax.jit
def _run(x, table_t):
    mesh = plsc.VectorSubcoreMesh(core_axis_name="c", subcore_axis_name="s")
    return pl.kernel(
        _gather_body,
        out_type=jax.ShapeDtypeStruct((D, B), jnp.float32),
        mesh=mesh,
        scratch_types=[
            pltpu.VMEM((B_PER_W,), jnp.int32),
            pltpu.VMEM((L, 4, 8, BLK), jnp.float32),
            pltpu.VMEM((D, B_PER_W), jnp.float32),
            pltpu.SemaphoreType.DMA((L,)),
        ],
        compiler_params=pltpu.CompilerParams(
            use_tc_tiling_on_sc=True,
            needs_layout_passes=False,
            disable_bounds_checks=True,
        ),
    )(table_t, x)


def kernel(x, table):
    out_t = _run(x.astype(jnp.int32), table.T.reshape(4, 8, N_ROWS))
    return out_t.T
